# baseline (speedup 1.0000x reference)
import jax
import jax.numpy as jnp
from jax import lax
from jax.experimental import pallas as pl
from jax.experimental.pallas import tpu as pltpu

_CW = 1024


def _beam_step_body(lp_ref, blps_ref, seq_ref, state_ref,
                    vals_ref, seq_out_ref, blp_ref, state_out_ref, scr_ref):
    K = lp_ref.shape[1]
    V = lp_ref.shape[2]
    T = seq_ref.shape[2]
    NC, CW = scr_ref.shape[0], scr_ref.shape[2]

    neg_mask = jnp.float32(-1e9)
    neg_inf = jnp.float32(-jnp.inf)
    big = jnp.int32(2**31 - 1)

    blps = blps_ref[0]
    prev = seq_ref[0, :, T - 1:T]

    for c in range(NC):
        lo = c * CW
        w = min(CW, V - lo)
        sub = lp_ref[0, :, lo:lo + w]
        iota_c = lax.broadcasted_iota(jnp.int32, (K, w), 1) + lo
        candc = jnp.where(iota_c == prev, neg_mask, sub) + blps
        if w < CW:
            candc = jnp.concatenate(
                [candc, jnp.full((K, CW - w), neg_inf, jnp.float32)], axis=1)
        scr_ref[c] = candc

    scr = scr_ref[...]
    M = jnp.max(scr, axis=2)

    iota_nc2 = lax.broadcasted_iota(jnp.int32, (NC, K), 0)
    iota_k2 = lax.broadcasted_iota(jnp.int32, (NC, K), 1)
    keymat = iota_k2 * NC + iota_nc2
    iota_k3c = lax.broadcasted_iota(jnp.int32, (1, K, CW), 1)
    iota_cw3c = lax.broadcasted_iota(jnp.int32, (1, K, CW), 2)
    picks = []
    for _ in range(K):
        m = jnp.max(M)
        key = jnp.min(jnp.where(M == m, keymat, big))
        k_sel = key // NC
        ci = key - k_sel * NC
        chunk = scr_ref[pl.ds(ci, 1)]
        linc = iota_k3c * V + ci * CW + iota_cw3c
        a = jnp.min(jnp.where((chunk == m) & (iota_k3c == k_sel), linc, big))
        r = a // V
        v = a - r * V
        picks.append((m, r, v))
        chunk = jnp.where(linc == a, neg_inf, chunk)
        scr_ref[pl.ds(ci, 1)] = chunk
        mc = jnp.max(chunk, axis=2)
        M = jnp.where(iota_nc2 == ci, mc, M)

    iota_kr = lax.broadcasted_iota(jnp.int32, (K, K), 0)
    iota_kc = lax.broadcasted_iota(jnp.int32, (K, K), 1)
    iota_k1 = lax.broadcasted_iota(jnp.int32, (K, 1), 0)
    perm = jnp.zeros((K, K), jnp.float32)
    psel = jnp.zeros((K, 1), jnp.int32)
    for j, (m, r, v) in enumerate(picks):
        vals_ref[0, j:j + 1, :] = jnp.full((1, 1), m, jnp.float32)
        seq_out_ref[0, j:j + 1, :T] = seq_ref[0, pl.ds(r, 1), :]
        seq_out_ref[0, j:j + 1, T:] = jnp.full((1, 1), v, jnp.int32)
        state_out_ref[:, 0, j:j + 1, :] = state_ref[:, 0, pl.ds(r, 1), :]
        prev_j = seq_ref[0, pl.ds(r, 1), T - 1:T]
        perm = jnp.where((iota_kr == j) & (iota_kc == r),
                         jnp.float32(1.0), perm)
        psel = jnp.where(iota_k1 == j, prev_j, psel)
    g = lax.dot_general(perm, lp_ref[0], (((1,), (0,)), ((), ())),
                        precision=lax.Precision.HIGHEST,
                        preferred_element_type=jnp.float32)
    iota_v2 = lax.broadcasted_iota(jnp.int32, (K, V), 1)
    blp_ref[0] = jnp.where(iota_v2 == psel, neg_mask, g)


def kernel(logprobs, beam_logprobs_sum, state, beam_seq):
    B, K, V = logprobs.shape
    T = beam_seq.shape[2]
    D = state.shape[-1]
    NC = -(-V // _CW)
    blps3 = beam_logprobs_sum.reshape(B, K, 1)
    state4 = state.reshape(state.shape[0], B, K, D)

    out_shape = [
        jax.ShapeDtypeStruct((B, K, 1), jnp.float32),
        jax.ShapeDtypeStruct((B, K, T + 1), jnp.int32),
        jax.ShapeDtypeStruct((B, K, V), jnp.float32),
        jax.ShapeDtypeStruct((state.shape[0], B, K, D), jnp.float32),
    ]
    in_specs = [
        pl.BlockSpec((1, K, V), lambda b: (b, 0, 0)),
        pl.BlockSpec((1, K, 1), lambda b: (b, 0, 0)),
        pl.BlockSpec((1, K, T), lambda b: (b, 0, 0)),
        pl.BlockSpec((state.shape[0], 1, K, D), lambda b: (0, b, 0, 0)),
    ]
    out_specs = [
        pl.BlockSpec((1, K, 1), lambda b: (b, 0, 0)),
        pl.BlockSpec((1, K, T + 1), lambda b: (b, 0, 0)),
        pl.BlockSpec((1, K, V), lambda b: (b, 0, 0)),
        pl.BlockSpec((state.shape[0], 1, K, D), lambda b: (0, b, 0, 0)),
    ]
    vals, new_seq, beam_lp, new_state = pl.pallas_call(
        _beam_step_body,
        grid=(B,),
        in_specs=in_specs,
        out_specs=out_specs,
        out_shape=out_shape,
        scratch_shapes=[pltpu.VMEM((NC, K, _CW), jnp.float32)],
    )(logprobs, blps3, beam_seq, state4)

    return (vals.reshape(B, K), new_seq, beam_lp,
            new_state.reshape(state.shape[0], B * K, D))


# speedup vs baseline: 1.0868x; 1.0868x over previous
import jax
import jax.numpy as jnp
from jax import lax
from jax.experimental import pallas as pl
from jax.experimental.pallas import tpu as pltpu

_CW = 1024


def _beam_step_body(lp_ref, blps_ref, seq_ref, state_ref,
                    vals_ref, seq_out_ref, blp_ref, state_out_ref, scr_ref):
    K = lp_ref.shape[1]
    V = lp_ref.shape[2]
    T = seq_ref.shape[2]
    NC, CW = scr_ref.shape[0], scr_ref.shape[2]

    neg_mask = jnp.float32(-1e9)
    neg_inf = jnp.float32(-jnp.inf)
    big = jnp.int32(2**31 - 1)

    blps = blps_ref[0]
    prev = seq_ref[0, :, T - 1:T]

    for c in range(NC):
        lo = c * CW
        w = min(CW, V - lo)
        sub = lp_ref[0, :, lo:lo + w]
        iota_c = lax.broadcasted_iota(jnp.int32, (K, w), 1) + lo
        candc = jnp.where(iota_c == prev, neg_mask, sub) + blps
        if w < CW:
            candc = jnp.concatenate(
                [candc, jnp.full((K, CW - w), neg_inf, jnp.float32)], axis=1)
        scr_ref[c] = candc

    scr = scr_ref[...]
    M = jnp.max(scr, axis=2)

    iota_nc2 = lax.broadcasted_iota(jnp.int32, (NC, K), 0)
    iota_k2 = lax.broadcasted_iota(jnp.int32, (NC, K), 1)
    keymat = iota_k2 * NC + iota_nc2
    iota_k3c = lax.broadcasted_iota(jnp.int32, (1, K, CW), 1)
    iota_cw3c = lax.broadcasted_iota(jnp.int32, (1, K, CW), 2)
    picks = []
    for _ in range(K):
        m = jnp.max(M)
        key = jnp.min(jnp.where(M == m, keymat, big))
        k_sel = key // NC
        ci = key - k_sel * NC
        chunk = scr_ref[pl.ds(ci, 1)]
        linc = iota_k3c * V + ci * CW + iota_cw3c
        a = jnp.min(jnp.where((chunk == m) & (iota_k3c == k_sel), linc, big))
        r = a // V
        v = a - r * V
        picks.append((m, r, v))
        chunk = jnp.where(linc == a, neg_inf, chunk)
        scr_ref[pl.ds(ci, 1)] = chunk
        mc = jnp.max(chunk, axis=2)
        M = jnp.where(iota_nc2 == ci, mc, M)

    iota_v1 = lax.broadcasted_iota(jnp.int32, (1, V), 1)
    for j, (m, r, v) in enumerate(picks):
        vals_ref[0, j:j + 1, :] = jnp.full((1, 1), m, jnp.float32)
        seq_out_ref[0, j:j + 1, :T] = seq_ref[0, pl.ds(r, 1), :]
        seq_out_ref[0, j:j + 1, T:] = jnp.full((1, 1), v, jnp.int32)
        state_out_ref[:, 0, j:j + 1, :] = state_ref[:, 0, pl.ds(r, 1), :]
        prev_j = seq_ref[0, pl.ds(r, 1), T - 1:T]
        row = lp_ref[0, pl.ds(r, 1), :]
        blp_ref[0, j:j + 1, :] = jnp.where(iota_v1 == prev_j, neg_mask, row)


def kernel(logprobs, beam_logprobs_sum, state, beam_seq):
    B, K, V = logprobs.shape
    T = beam_seq.shape[2]
    D = state.shape[-1]
    NC = -(-V // _CW)
    blps3 = beam_logprobs_sum.reshape(B, K, 1)
    state4 = state.reshape(state.shape[0], B, K, D)

    out_shape = [
        jax.ShapeDtypeStruct((B, K, 1), jnp.float32),
        jax.ShapeDtypeStruct((B, K, T + 1), jnp.int32),
        jax.ShapeDtypeStruct((B, K, V), jnp.float32),
        jax.ShapeDtypeStruct((state.shape[0], B, K, D), jnp.float32),
    ]
    in_specs = [
        pl.BlockSpec((1, K, V), lambda b: (b, 0, 0)),
        pl.BlockSpec((1, K, 1), lambda b: (b, 0, 0)),
        pl.BlockSpec((1, K, T), lambda b: (b, 0, 0)),
        pl.BlockSpec((state.shape[0], 1, K, D), lambda b: (0, b, 0, 0)),
    ]
    out_specs = [
        pl.BlockSpec((1, K, 1), lambda b: (b, 0, 0)),
        pl.BlockSpec((1, K, T + 1), lambda b: (b, 0, 0)),
        pl.BlockSpec((1, K, V), lambda b: (b, 0, 0)),
        pl.BlockSpec((state.shape[0], 1, K, D), lambda b: (0, b, 0, 0)),
    ]
    vals, new_seq, beam_lp, new_state = pl.pallas_call(
        _beam_step_body,
        grid=(B,),
        in_specs=in_specs,
        out_specs=out_specs,
        out_shape=out_shape,
        scratch_shapes=[pltpu.VMEM((NC, K, _CW), jnp.float32)],
    )(logprobs, blps3, beam_seq, state4)

    return (vals.reshape(B, K), new_seq, beam_lp,
            new_state.reshape(state.shape[0], B * K, D))


# speedup vs baseline: 1.1115x; 1.0228x over previous
import jax
import jax.numpy as jnp
from jax import lax
from jax.experimental import pallas as pl
from jax.experimental.pallas import tpu as pltpu

_CW = 1024


def _beam_step_body(lp_ref, blps_ref, seq_ref, state_ref,
                    vals_ref, seq_out_ref, blp_ref, state_out_ref, scr_ref):
    K = lp_ref.shape[1]
    V = lp_ref.shape[2]
    T = seq_ref.shape[2]
    NC, CW = scr_ref.shape[0], scr_ref.shape[2]

    neg_mask = jnp.float32(-1e9)
    neg_inf = jnp.float32(-jnp.inf)
    big = jnp.int32(2**31 - 1)

    blps = blps_ref[0]
    prev = seq_ref[0, :, T - 1:T]

    for c in range(NC):
        lo = c * CW
        w = min(CW, V - lo)
        sub = lp_ref[0, :, lo:lo + w]
        iota_c = lax.broadcasted_iota(jnp.int32, (K, w), 1) + lo
        candc = jnp.where(iota_c == prev, neg_mask, sub) + blps
        if w < CW:
            candc = jnp.concatenate(
                [candc, jnp.full((K, CW - w), neg_inf, jnp.float32)], axis=1)
        scr_ref[c] = candc

    scr = scr_ref[...]
    iota_nc3 = lax.broadcasted_iota(jnp.int32, (NC, K, CW), 0)
    iota_k3 = lax.broadcasted_iota(jnp.int32, (NC, K, CW), 1)
    iota_cw3 = lax.broadcasted_iota(jnp.int32, (NC, K, CW), 2)
    lin3 = iota_k3 * V + iota_nc3 * CW + iota_cw3
    M = jnp.max(scr, axis=2)
    A = jnp.min(jnp.where(scr == M[:, :, None], lin3, big), axis=2)

    iota_nc2 = lax.broadcasted_iota(jnp.int32, (NC, K), 0)
    picks = []
    for _ in range(K):
        m = jnp.max(M)
        a = jnp.min(jnp.where(M == m, A, big))
        r = a // V
        v = a - r * V
        picks.append((m, r, v))
        ci = v // CW
        chunk = scr_ref[pl.ds(ci, 1)]
        linc = (lax.broadcasted_iota(jnp.int32, (1, K, CW), 1) * V + ci * CW
                + lax.broadcasted_iota(jnp.int32, (1, K, CW), 2))
        chunk = jnp.where(linc == a, neg_inf, chunk)
        scr_ref[pl.ds(ci, 1)] = chunk
        mc = jnp.max(chunk, axis=2)
        ac = jnp.min(jnp.where(chunk == mc[:, :, None], linc, big), axis=2)
        hit = iota_nc2 == ci
        M = jnp.where(hit, mc, M)
        A = jnp.where(hit, ac, A)

    iota_v1 = lax.broadcasted_iota(jnp.int32, (1, V), 1)
    for j, (m, r, v) in enumerate(picks):
        vals_ref[0, j:j + 1, :] = jnp.full((1, 1), m, jnp.float32)
        seq_out_ref[0, j:j + 1, :T] = seq_ref[0, pl.ds(r, 1), :]
        seq_out_ref[0, j:j + 1, T:] = jnp.full((1, 1), v, jnp.int32)
        state_out_ref[:, 0, j:j + 1, :] = state_ref[:, 0, pl.ds(r, 1), :]
        prev_j = seq_ref[0, pl.ds(r, 1), T - 1:T]
        row = lp_ref[0, pl.ds(r, 1), :]
        blp_ref[0, j:j + 1, :] = jnp.where(iota_v1 == prev_j, neg_mask, row)


def kernel(logprobs, beam_logprobs_sum, state, beam_seq):
    B, K, V = logprobs.shape
    T = beam_seq.shape[2]
    D = state.shape[-1]
    NC = -(-V // _CW)
    blps3 = beam_logprobs_sum.reshape(B, K, 1)
    state4 = state.reshape(state.shape[0], B, K, D)

    out_shape = [
        jax.ShapeDtypeStruct((B, K, 1), jnp.float32),
        jax.ShapeDtypeStruct((B, K, T + 1), jnp.int32),
        jax.ShapeDtypeStruct((B, K, V), jnp.float32),
        jax.ShapeDtypeStruct((state.shape[0], B, K, D), jnp.float32),
    ]
    in_specs = [
        pl.BlockSpec((1, K, V), lambda b: (b, 0, 0)),
        pl.BlockSpec((1, K, 1), lambda b: (b, 0, 0)),
        pl.BlockSpec((1, K, T), lambda b: (b, 0, 0)),
        pl.BlockSpec((state.shape[0], 1, K, D), lambda b: (0, b, 0, 0)),
    ]
    out_specs = [
        pl.BlockSpec((1, K, 1), lambda b: (b, 0, 0)),
        pl.BlockSpec((1, K, T + 1), lambda b: (b, 0, 0)),
        pl.BlockSpec((1, K, V), lambda b: (b, 0, 0)),
        pl.BlockSpec((state.shape[0], 1, K, D), lambda b: (0, b, 0, 0)),
    ]
    vals, new_seq, beam_lp, new_state = pl.pallas_call(
        _beam_step_body,
        grid=(B,),
        in_specs=in_specs,
        out_specs=out_specs,
        out_shape=out_shape,
        scratch_shapes=[pltpu.VMEM((NC, K, _CW), jnp.float32)],
    )(logprobs, blps3, beam_seq, state4)

    return (vals.reshape(B, K), new_seq, beam_lp,
            new_state.reshape(state.shape[0], B * K, D))
